# baseline (speedup 1.0000x reference)
import functools

import jax
import jax.numpy as jnp
from jax.experimental import pallas as pl
from jax.experimental.pallas import tpu as pltpu


def _gcn_body(x_ref, w0_ref, b0_ref, w1_ref, b1_ref, adj_a_ref, adj_b_ref,
              out_ref, y0_ref, h_ref, y1_ref, stats_ref, *,
              nb, block_m, n):
    i = pl.program_id(0)
    hm = block_m // 2

    @pl.when(i == 0)
    def _init():
        y0_ref[...] = jnp.dot(x_ref[...], w0_ref[...],
                              preferred_element_type=jnp.float32
                              ).astype(jnp.bfloat16)
        stats_ref[...] = jnp.zeros_like(stats_ref)

    @pl.when(i < nb)
    def _layer1():
        ht = jnp.dot(adj_a_ref[...].astype(jnp.bfloat16), y0_ref[...],
                     preferred_element_type=jnp.float32)
        hb = jnp.dot(adj_b_ref[...].astype(jnp.bfloat16), y0_ref[...],
                     preferred_element_type=jnp.float32)
        h = jnp.concatenate([ht, hb], axis=0)
        h = jnp.maximum(h + b0_ref[...], 0.0)
        h_ref[pl.ds(i * block_m, block_m), :] = h
        stats_ref[0:1, :] += jnp.sum(h, axis=0, keepdims=True)
        stats_ref[1:2, :] += jnp.sum(h * h, axis=0, keepdims=True)

    @pl.when(i == nb)
    def _bn_project():
        mean = stats_ref[0:1, :] / n
        var = stats_ref[1:2, :] / n - mean * mean
        scale = jax.lax.rsqrt(var + 1e-5)
        hn = (h_ref[...] - mean) * scale
        y1_ref[...] = jnp.dot(hn, w1_ref[...],
                              preferred_element_type=jnp.float32
                              ).astype(jnp.bfloat16)

    @pl.when(i >= nb)
    def _layer2():
        ot = jnp.dot(adj_a_ref[...].astype(jnp.bfloat16), y1_ref[...],
                     preferred_element_type=jnp.float32)
        ob = jnp.dot(adj_b_ref[...].astype(jnp.bfloat16), y1_ref[...],
                     preferred_element_type=jnp.float32)
        out_ref[0:hm, :] = ot + b1_ref[...]
        out_ref[hm:block_m, :] = ob + b1_ref[...]


@functools.partial(jax.jit, static_argnames=("block_m",))
def _gcn(x, adj, W0, b0, W1, b1, block_m=400):
    n, d = x.shape
    h_dim = W0.shape[1]
    o_dim = W1.shape[1]
    nb = n // block_m
    hm = block_m // 2

    out = pl.pallas_call(
        functools.partial(_gcn_body, nb=nb, block_m=block_m, n=n),
        grid=(2 * nb,),
        in_specs=[
            pl.BlockSpec((n, d), lambda i: (0, 0)),
            pl.BlockSpec((d, h_dim), lambda i: (0, 0)),
            pl.BlockSpec((1, h_dim), lambda i: (0, 0)),
            pl.BlockSpec((h_dim, o_dim), lambda i: (0, 0)),
            pl.BlockSpec((1, o_dim), lambda i: (0, 0)),
            pl.BlockSpec((hm, n),
                         lambda i: (2 * jax.lax.rem(i, nb), 0)),
            pl.BlockSpec((hm, n),
                         lambda i: (2 * jax.lax.rem(i, nb) + 1, 0)),
        ],
        out_specs=pl.BlockSpec((block_m, o_dim),
                               lambda i: (jnp.maximum(i - nb, 0), 0)),
        out_shape=jax.ShapeDtypeStruct((n, o_dim), jnp.float32),
        scratch_shapes=[
            pltpu.VMEM((n, h_dim), jnp.bfloat16),
            pltpu.VMEM((n, h_dim), jnp.float32),
            pltpu.VMEM((n, o_dim), jnp.bfloat16),
            pltpu.VMEM((8, h_dim), jnp.float32),
        ],
    )(x, W0, b0.reshape(1, h_dim), W1, b1.reshape(1, o_dim), adj, adj)
    return out


def kernel(x, adj, W0, b0, W1, b1):
    return _gcn(x, adj, W0, b0, W1, b1)


# speedup vs baseline: 1.0072x; 1.0072x over previous
import functools

import jax
import jax.numpy as jnp
from jax.experimental import pallas as pl
from jax.experimental.pallas import tpu as pltpu


def _gcn_body(x_ref, w0_ref, b0_ref, w1_ref, b1_ref, adj_ref,
              out_ref, y0_ref, h_ref, y1_ref, *, nb, block_m, n):
    i = pl.program_id(0)

    @pl.when(i == 0)
    def _init():
        y0_ref[...] = jnp.dot(x_ref[...], w0_ref[...],
                              preferred_element_type=jnp.float32
                              ).astype(jnp.bfloat16)

    @pl.when(i < nb)
    def _layer1():
        h = jnp.dot(adj_ref[...].astype(jnp.bfloat16), y0_ref[...],
                    preferred_element_type=jnp.float32)
        h = jnp.maximum(h + b0_ref[...], 0.0)
        h_ref[pl.ds(i * block_m, block_m), :] = h

    @pl.when(i == nb)
    def _bn_project():
        h = h_ref[...]
        mean = jnp.sum(h, axis=0, keepdims=True) / n
        var = jnp.sum(h * h, axis=0, keepdims=True) / n - mean * mean
        scale = jax.lax.rsqrt(var + 1e-5)
        hn = (h - mean) * scale
        y1_ref[...] = jnp.dot(hn.astype(jnp.bfloat16), w1_ref[...],
                              preferred_element_type=jnp.float32
                              ).astype(jnp.bfloat16)

    @pl.when(i >= nb)
    def _layer2():
        o = jnp.dot(adj_ref[...].astype(jnp.bfloat16), y1_ref[...],
                    preferred_element_type=jnp.float32)
        out_ref[...] = o + b1_ref[...]


@functools.partial(jax.jit, static_argnames=("block_m",))
def _gcn(x, adj, W0, b0, W1, b1, block_m=400):
    n, d = x.shape
    h_dim = W0.shape[1]
    o_dim = W1.shape[1]
    nb = n // block_m

    out = pl.pallas_call(
        functools.partial(_gcn_body, nb=nb, block_m=block_m, n=n),
        grid=(2 * nb,),
        in_specs=[
            pl.BlockSpec((n, d), lambda i: (0, 0)),
            pl.BlockSpec((d, h_dim), lambda i: (0, 0)),
            pl.BlockSpec((1, h_dim), lambda i: (0, 0)),
            pl.BlockSpec((h_dim, o_dim), lambda i: (0, 0)),
            pl.BlockSpec((1, o_dim), lambda i: (0, 0)),
            pl.BlockSpec((block_m, n),
                         lambda i: (jax.lax.rem(i, nb), 0)),
        ],
        out_specs=pl.BlockSpec((block_m, o_dim),
                               lambda i: (jnp.maximum(i - nb, 0), 0)),
        out_shape=jax.ShapeDtypeStruct((n, o_dim), jnp.float32),
        scratch_shapes=[
            pltpu.VMEM((n, h_dim), jnp.bfloat16),
            pltpu.VMEM((n, h_dim), jnp.float32),
            pltpu.VMEM((n, o_dim), jnp.bfloat16),
        ],
    )(x, W0, b0.reshape(1, h_dim), W1, b1.reshape(1, o_dim), adj)
    return out


def kernel(x, adj, W0, b0, W1, b1):
    return _gcn(x, adj, W0, b0, W1, b1)


# speedup vs baseline: 1.0198x; 1.0126x over previous
import functools

import jax
import jax.numpy as jnp
from jax.experimental import pallas as pl
from jax.experimental.pallas import tpu as pltpu


def _gcn_body(x_ref, w0_ref, b0_ref, w1_ref, b1_ref, adj_ref,
              out_ref, xb_ref, hb_ref, stats_ref, w1s_ref, *,
              nb, block_m, n, h_dim):
    i = pl.program_id(0)

    @pl.when(i == 0)
    def _init():
        xb_ref[...] = x_ref[...].astype(jnp.bfloat16)
        stats_ref[...] = jnp.zeros_like(stats_ref)
        col = jax.lax.broadcasted_iota(jnp.int32, (n, h_dim), 1)
        hb_ref[:, h_dim:] = jnp.where(col == 0, 1.0, 0.0).astype(jnp.bfloat16)

    @pl.when(i < nb)
    def _layer1():
        u = jnp.dot(adj_ref[...].astype(jnp.bfloat16), xb_ref[...],
                    preferred_element_type=jnp.float32)
        h = jnp.dot(u, w0_ref[...], preferred_element_type=jnp.float32)
        h = jnp.maximum(h + b0_ref[...], 0.0)
        hb_ref[pl.ds(i * block_m, block_m), 0:h_dim] = h.astype(jnp.bfloat16)
        stats_ref[0:1, :] += jnp.sum(h, axis=0, keepdims=True)
        stats_ref[1:2, :] += jnp.sum(h * h, axis=0, keepdims=True)

    @pl.when(i == nb)
    def _bn_finalize():
        mean = stats_ref[0:1, :] / n
        var = stats_ref[1:2, :] / n - mean * mean
        scale = jax.lax.rsqrt(var + 1e-5)
        w1s_ref[...] = scale.reshape(h_dim, 1) * w1_ref[...]
        stats_ref[2:3, :] = jnp.dot(mean * scale, w1_ref[...],
                                    preferred_element_type=jnp.float32)

    @pl.when(i >= nb)
    def _layer2():
        g = jnp.dot(adj_ref[...].astype(jnp.bfloat16), hb_ref[...],
                    preferred_element_type=jnp.float32)
        u = g[:, 0:h_dim]
        r = g[:, h_dim:h_dim + 1]
        o = jnp.dot(u, w1s_ref[...], preferred_element_type=jnp.float32)
        out_ref[...] = o - r * stats_ref[2:3, :] + b1_ref[...]


@functools.partial(jax.jit, static_argnames=("block_m",))
def _gcn(x, adj, W0, b0, W1, b1, block_m=400):
    n, d = x.shape
    h_dim = W0.shape[1]
    o_dim = W1.shape[1]
    nb = n // block_m

    out = pl.pallas_call(
        functools.partial(_gcn_body, nb=nb, block_m=block_m, n=n,
                          h_dim=h_dim),
        grid=(2 * nb,),
        in_specs=[
            pl.BlockSpec((n, d), lambda i: (0, 0)),
            pl.BlockSpec((d, h_dim), lambda i: (0, 0)),
            pl.BlockSpec((1, h_dim), lambda i: (0, 0)),
            pl.BlockSpec((h_dim, o_dim), lambda i: (0, 0)),
            pl.BlockSpec((1, o_dim), lambda i: (0, 0)),
            pl.BlockSpec((block_m, n),
                         lambda i: (jax.lax.rem(i, nb), 0)),
        ],
        out_specs=pl.BlockSpec((block_m, o_dim),
                               lambda i: (jnp.maximum(i - nb, 0), 0)),
        out_shape=jax.ShapeDtypeStruct((n, o_dim), jnp.float32),
        scratch_shapes=[
            pltpu.VMEM((n, d), jnp.bfloat16),
            pltpu.VMEM((n, 2 * h_dim), jnp.bfloat16),
            pltpu.VMEM((8, h_dim), jnp.float32),
            pltpu.VMEM((h_dim, o_dim), jnp.float32),
        ],
    )(x, W0, b0.reshape(1, h_dim), W1, b1.reshape(1, o_dim), adj)
    return out


def kernel(x, adj, W0, b0, W1, b1):
    return _gcn(x, adj, W0, b0, W1, b1)
